# Initial kernel scaffold; baseline (speedup 1.0000x reference)
import jax
import jax.numpy as jnp
from jax.experimental import pallas as pl


def kernel(x, A, W1, b1, p1, W2, b2, p2):
    raise NotImplementedError("write your pallas kernel here")


# speedup vs baseline: 1.4005x; 1.4005x over previous
import functools
import math

import jax
import jax.numpy as jnp
from jax.experimental import pallas as pl

RB = 512
TT = 512

_DOT = dict(preferred_element_type=jnp.float32,
            precision=jax.lax.Precision.HIGHEST)


def _pass1_kernel(x_ref, w1_ref, a_ref, deg_ref, diag_ref, xw_ref):
    r = pl.program_id(1)
    a = a_ref[0]
    colsum = jnp.sum(a, axis=0)

    @pl.when(r == 0)
    def _init():
        deg_ref[0, 0] = jnp.zeros_like(deg_ref[0, 0])

    deg_ref[0, 0] += colsum

    rb = a.shape[0]
    jidx = jax.lax.broadcasted_iota(jnp.int32, (rb, a.shape[1]), 1)
    iidx = jax.lax.broadcasted_iota(jnp.int32, (rb, a.shape[1]), 0) + r * rb
    diag_ref[0, 0, pl.ds(r * rb, rb)] = jnp.sum(
        jnp.where(jidx == iidx, a, 0.0), axis=1)

    xw_ref[0] = jax.lax.dot_general(x_ref[0], w1_ref[...],
                                    (((1,), (0,)), ((), ())), **_DOT)


def _pass2_kernel(a_ref, xw_ref, deg_ref, diag_ref, b1_ref, p1_ref,
                  h1_ref, s1_ref, *, nr):
    r = pl.program_id(1)
    rb = a_ref.shape[1]

    def _dis(deg_raw, diag):
        deg = deg_raw + jnp.where(diag == 0.0, 1.0, 0.0)
        return jnp.where(deg > 0.0,
                         jax.lax.rsqrt(jnp.where(deg > 0.0, deg, 1.0)), 0.0)

    sl = pl.ds(r * rb, rb)
    dis1_blk = _dis(deg_ref[0, 0, sl], diag_ref[0, 0, sl])
    u_blk = dis1_blk[:, None] * xw_ref[0, sl, :]

    @pl.when(r == 0)
    def _init():
        h1_ref[0] = jnp.zeros_like(h1_ref[0])

    h1_ref[0] += jax.lax.dot_general(a_ref[0], u_blk,
                                     (((0,), (0,)), ((), ())), **_DOT)

    @pl.when(r == nr - 1)
    def _fini():
        add1 = jnp.where(diag_ref[0, 0] == 0.0, 1.0, 0.0)
        dis1 = _dis(deg_ref[0, 0], diag_ref[0, 0])
        u_all = dis1[:, None] * xw_ref[0]
        acc = h1_ref[0] + add1[:, None] * u_all
        h1 = jnp.maximum(dis1[:, None] * acc + b1_ref[0][None, :], 0.0)
        h1_ref[0] = h1
        p = p1_ref[0]
        pn = jax.lax.rsqrt(jnp.sum(p * p))
        s1_ref[0, 0] = jnp.tanh(jnp.sum(h1 * p[None, :], axis=1) * pn)


def _rank1_kernel(s1_ref, h1_ref, w2_ref, rank_ref, m1_ref, zw2_ref, *, k1):
    t = pl.program_id(1)
    s_all = s1_ref[0, 0]
    tt = h1_ref.shape[1]
    s_i = s1_ref[0, 0, pl.ds(t * tt, tt)]
    n = s_all.shape[0]
    jidx = jax.lax.broadcasted_iota(jnp.int32, (tt, n), 1)
    iidx = jax.lax.broadcasted_iota(jnp.int32, (tt, n), 0) + t * tt
    gt = (s_all[None, :] > s_i[:, None]).astype(jnp.float32)
    tie = jnp.where((s_all[None, :] == s_i[:, None]) & (jidx < iidx), 1.0, 0.0)
    rank = jnp.sum(gt + tie, axis=1)
    m1 = (rank < k1).astype(jnp.float32)
    rank_ref[0, 0, pl.ds(t * tt, tt)] = rank
    m1_ref[0, 0, pl.ds(t * tt, tt)] = m1
    z = h1_ref[0] * (s_i * m1)[:, None]
    zw2_ref[0] = jax.lax.dot_general(z, w2_ref[...],
                                     (((1,), (0,)), ((), ())), **_DOT)


def _pass3_kernel(a_ref, m1_ref, deg2_ref):
    r = pl.program_id(1)
    rb = a_ref.shape[1]
    m_blk = m1_ref[0, 0, pl.ds(r * rb, rb)]

    @pl.when(r == 0)
    def _init():
        deg2_ref[0, 0] = jnp.zeros_like(deg2_ref[0, 0])

    deg2_ref[0, 0] += jax.lax.dot_general(
        m_blk[None, :], a_ref[0], (((1,), (0,)), ((), ())), **_DOT)[0]


def _pass4_kernel(a_ref, zw2_ref, deg2_ref, diag_ref, m1_ref, b2_ref, p2_ref,
                  h2_ref, s2_ref, *, nr):
    r = pl.program_id(1)
    rb = a_ref.shape[1]

    def _gscale(deg2_raw, diag, m1):
        deg2 = deg2_raw + jnp.where(diag == 0.0, 1.0, 0.0)
        dis2 = jnp.where(deg2 > 0.0,
                         jax.lax.rsqrt(jnp.where(deg2 > 0.0, deg2, 1.0)), 0.0)
        return m1 * dis2, dis2

    sl = pl.ds(r * rb, rb)
    gscale_blk, _ = _gscale(deg2_ref[0, 0, sl], diag_ref[0, 0, sl],
                            m1_ref[0, 0, sl])
    g_blk = gscale_blk[:, None] * zw2_ref[0, sl, :]

    @pl.when(r == 0)
    def _init():
        h2_ref[0] = jnp.zeros_like(h2_ref[0])

    h2_ref[0] += jax.lax.dot_general(a_ref[0], g_blk,
                                     (((0,), (0,)), ((), ())), **_DOT)

    @pl.when(r == nr - 1)
    def _fini():
        add1 = jnp.where(diag_ref[0, 0] == 0.0, 1.0, 0.0)
        gscale, dis2 = _gscale(deg2_ref[0, 0], diag_ref[0, 0], m1_ref[0, 0])
        g_all = gscale[:, None] * zw2_ref[0]
        acc = h2_ref[0] + add1[:, None] * g_all
        h2 = jnp.maximum(dis2[:, None] * acc + b2_ref[0][None, :], 0.0)
        h2_ref[0] = h2
        p = p2_ref[0]
        pn = jax.lax.rsqrt(jnp.sum(p * p))
        s2_ref[0, 0] = jnp.tanh(jnp.sum(h2 * p[None, :], axis=1) * pn)


def _rank2_kernel(s2_ref, rank1_ref, m1_ref, h2_ref, rank2_ref, v_ref, *, k2):
    t = pl.program_id(1)
    s_all = s2_ref[0, 0]
    r1_all = rank1_ref[0, 0]
    m_all = m1_ref[0, 0]
    tt = h2_ref.shape[1]
    sl = pl.ds(t * tt, tt)
    s_i = s2_ref[0, 0, sl]
    r1_i = rank1_ref[0, 0, sl]
    m_i = m1_ref[0, 0, sl]
    gt = (s_all[None, :] > s_i[:, None]).astype(jnp.float32)
    tie = jnp.where((s_all[None, :] == s_i[:, None])
                    & (r1_all[None, :] < r1_i[:, None]), 1.0, 0.0)
    rank2 = jnp.sum(m_all[None, :] * (gt + tie), axis=1)
    valid = m_i * (rank2 < k2).astype(jnp.float32)
    rank2_ref[0, 0, sl] = jnp.where(valid > 0.0, rank2, -1.0)
    v_ref[0] = h2_ref[0] * (s_i * valid)[:, None]


def _scatter_kernel(rank2_ref, v_ref, out_ref):
    q = pl.program_id(1)
    tq = out_ref.shape[1]
    r2 = rank2_ref[0, 0]
    n = r2.shape[0]
    qidx = (jax.lax.broadcasted_iota(jnp.int32, (n, tq), 1)
            + q * tq).astype(jnp.float32)
    p = jnp.where(r2[:, None] == qidx, 1.0, 0.0)
    out_ref[0] = jax.lax.dot_general(p, v_ref[0],
                                     (((0,), (0,)), ((), ())), **_DOT)


def kernel(x, A, W1, b1, p1, W2, b2, p2):
    B, N, D0 = x.shape
    D1 = W1.shape[1]
    D2 = W2.shape[1]
    k1 = int(math.ceil(0.8 * N))
    k2 = int(math.ceil(0.5 * k1))
    nr = N // RB
    nt = N // TT
    f32 = jnp.float32
    b1r, p1r = b1.reshape(1, D1), p1.reshape(1, D1)
    b2r, p2r = b2.reshape(1, D2), p2.reshape(1, D2)

    full_n = pl.BlockSpec((1, 1, N), lambda b, r: (b, 0, 0))
    full_nd1 = pl.BlockSpec((1, N, D1), lambda b, r: (b, 0, 0))
    full_nd2 = pl.BlockSpec((1, N, D2), lambda b, r: (b, 0, 0))
    a_blk = pl.BlockSpec((1, RB, N), lambda b, r: (b, r, 0))
    small1 = pl.BlockSpec((1, D1), lambda b, r: (0, 0))
    small2 = pl.BlockSpec((1, D2), lambda b, r: (0, 0))

    deg1raw, diag, xw1 = pl.pallas_call(
        _pass1_kernel,
        grid=(B, nr),
        in_specs=[pl.BlockSpec((1, RB, D0), lambda b, r: (b, r, 0)),
                  pl.BlockSpec((D0, D1), lambda b, r: (0, 0)),
                  a_blk],
        out_specs=[full_n, full_n, pl.BlockSpec((1, RB, D1),
                                                lambda b, r: (b, r, 0))],
        out_shape=[jax.ShapeDtypeStruct((B, 1, N), f32),
                   jax.ShapeDtypeStruct((B, 1, N), f32),
                   jax.ShapeDtypeStruct((B, N, D1), f32)],
    )(x, W1, A)

    h1, s1 = pl.pallas_call(
        functools.partial(_pass2_kernel, nr=nr),
        grid=(B, nr),
        in_specs=[a_blk, full_nd1, full_n, full_n, small1, small1],
        out_specs=[full_nd1, full_n],
        out_shape=[jax.ShapeDtypeStruct((B, N, D1), f32),
                   jax.ShapeDtypeStruct((B, 1, N), f32)],
    )(A, xw1, deg1raw, diag, b1r, p1r)

    rank1, m1, zw2 = pl.pallas_call(
        functools.partial(_rank1_kernel, k1=k1),
        grid=(B, nt),
        in_specs=[full_n,
                  pl.BlockSpec((1, TT, D1), lambda b, t: (b, t, 0)),
                  pl.BlockSpec((D1, D2), lambda b, t: (0, 0))],
        out_specs=[full_n, full_n,
                   pl.BlockSpec((1, TT, D2), lambda b, t: (b, t, 0))],
        out_shape=[jax.ShapeDtypeStruct((B, 1, N), f32),
                   jax.ShapeDtypeStruct((B, 1, N), f32),
                   jax.ShapeDtypeStruct((B, N, D2), f32)],
    )(s1, h1, W2)

    deg2raw = pl.pallas_call(
        _pass3_kernel,
        grid=(B, nr),
        in_specs=[a_blk, full_n],
        out_specs=full_n,
        out_shape=jax.ShapeDtypeStruct((B, 1, N), f32),
    )(A, m1)

    h2, s2 = pl.pallas_call(
        functools.partial(_pass4_kernel, nr=nr),
        grid=(B, nr),
        in_specs=[a_blk, full_nd2, full_n, full_n, full_n, small2, small2],
        out_specs=[full_nd2, full_n],
        out_shape=[jax.ShapeDtypeStruct((B, N, D2), f32),
                   jax.ShapeDtypeStruct((B, 1, N), f32)],
    )(A, zw2, deg2raw, diag, m1, b2r, p2r)

    rank2, v = pl.pallas_call(
        functools.partial(_rank2_kernel, k2=k2),
        grid=(B, nt),
        in_specs=[full_n, full_n, full_n,
                  pl.BlockSpec((1, TT, D2), lambda b, t: (b, t, 0))],
        out_specs=[full_n,
                   pl.BlockSpec((1, TT, D2), lambda b, t: (b, t, 0))],
        out_shape=[jax.ShapeDtypeStruct((B, 1, N), f32),
                   jax.ShapeDtypeStruct((B, N, D2), f32)],
    )(s2, rank1, m1, h2)

    out = pl.pallas_call(
        _scatter_kernel,
        grid=(B, nt),
        in_specs=[full_n, full_nd2],
        out_specs=pl.BlockSpec((1, TT, D2), lambda b, q: (b, q, 0)),
        out_shape=jax.ShapeDtypeStruct((B, N, D2), f32),
    )(rank2, v)
    return out


# speedup vs baseline: 2.0327x; 1.4514x over previous
import functools
import math

import jax
import jax.numpy as jnp
from jax.experimental import pallas as pl

RB = 512
TT = 512

_DOT = dict(preferred_element_type=jnp.float32,
            precision=jax.lax.Precision.HIGHEST)


def _dis_of(deg_raw, diag):
    deg = deg_raw + jnp.where(diag == 0.0, 1.0, 0.0)
    return jnp.where(deg > 0.0,
                     jax.lax.rsqrt(jnp.where(deg > 0.0, deg, 1.0)), 0.0)


def _pass1_kernel(x_ref, w1_ref, a_ref, deg_ref, diag_ref, xwt_ref):
    r = pl.program_id(1)
    a = a_ref[0]
    rb = a.shape[0]

    @pl.when(r == 0)
    def _init():
        deg_ref[0, 0] = jnp.zeros_like(deg_ref[0, 0])

    deg_ref[0, 0] += jnp.sum(a, axis=0)

    asq = a_ref[0, :, pl.ds(r * rb, rb)]
    ii = jax.lax.broadcasted_iota(jnp.int32, (rb, rb), 0)
    jj = jax.lax.broadcasted_iota(jnp.int32, (rb, rb), 1)
    diag_ref[0, 0, pl.ds(r * rb, rb)] = jnp.sum(
        jnp.where(ii == jj, asq, 0.0), axis=1)

    xw = jax.lax.dot_general(x_ref[0], w1_ref[...],
                             (((1,), (0,)), ((), ())), **_DOT)
    xwt_ref[0] = xw.T


def _pass2_kernel(a_ref, xwt_ref, deg_ref, diag_ref, b1_ref, p1_ref,
                  h1t_ref, s1_ref, *, nr):
    r = pl.program_id(1)
    rb = a_ref.shape[1]
    sl = pl.ds(r * rb, rb)
    dis1_blk = _dis_of(deg_ref[0, 0, sl], diag_ref[0, 0, sl])
    ut_blk = dis1_blk[None, :] * xwt_ref[0, :, sl]

    @pl.when(r == 0)
    def _init():
        h1t_ref[0] = jnp.zeros_like(h1t_ref[0])

    h1t_ref[0] += jax.lax.dot_general(ut_blk, a_ref[0],
                                      (((1,), (0,)), ((), ())), **_DOT)

    @pl.when(r == nr - 1)
    def _fini():
        add1 = jnp.where(diag_ref[0, 0] == 0.0, 1.0, 0.0)
        dis1 = _dis_of(deg_ref[0, 0], diag_ref[0, 0])
        ut_all = dis1[None, :] * xwt_ref[0]
        acc = h1t_ref[0] + add1[None, :] * ut_all
        h1t = jnp.maximum(dis1[None, :] * acc + b1_ref[0][:, None], 0.0)
        h1t_ref[0] = h1t
        p = p1_ref[0]
        pn = jax.lax.rsqrt(jnp.sum(p * p))
        s1_ref[0, 0] = jnp.tanh(jnp.sum(h1t * p[:, None], axis=0) * pn)


def _pass3_kernel(s1_ref, h1t_ref, w2_ref, a_ref,
                  rank_ref, m1_ref, zw2t_ref, deg2_ref, *, k1):
    r = pl.program_id(1)
    rb = a_ref.shape[1]
    sl = pl.ds(r * rb, rb)
    s_all = s1_ref[0, 0]
    s_i = s1_ref[0, 0, sl]
    n = s_all.shape[0]
    jidx = jax.lax.broadcasted_iota(jnp.int32, (rb, n), 1)
    iidx = jax.lax.broadcasted_iota(jnp.int32, (rb, n), 0) + r * rb
    gt = (s_all[None, :] > s_i[:, None]).astype(jnp.float32)
    tie = jnp.where((s_all[None, :] == s_i[:, None]) & (jidx < iidx), 1.0, 0.0)
    rank = jnp.sum(gt + tie, axis=1)
    m1 = (rank < k1).astype(jnp.float32)
    rank_ref[0, 0, sl] = rank
    m1_ref[0, 0, sl] = m1
    zt = h1t_ref[0, :, sl] * (s_i * m1)[None, :]
    zw2t_ref[0] = jax.lax.dot_general(w2_ref[...], zt,
                                      (((0,), (0,)), ((), ())), **_DOT)

    @pl.when(r == 0)
    def _init():
        deg2_ref[0, 0] = jnp.zeros_like(deg2_ref[0, 0])

    deg2_ref[0, 0] += jnp.sum(jnp.where(m1[:, None] > 0.0, a_ref[0], 0.0),
                              axis=0)


def _pass4_kernel(a_ref, zw2t_ref, deg2_ref, diag_ref, m1_ref, b2_ref, p2_ref,
                  h2t_ref, s2_ref, *, nr):
    r = pl.program_id(1)
    rb = a_ref.shape[1]
    sl = pl.ds(r * rb, rb)
    gs_blk = m1_ref[0, 0, sl] * _dis_of(deg2_ref[0, 0, sl], diag_ref[0, 0, sl])
    gt_blk = gs_blk[None, :] * zw2t_ref[0, :, sl]

    @pl.when(r == 0)
    def _init():
        h2t_ref[0] = jnp.zeros_like(h2t_ref[0])

    h2t_ref[0] += jax.lax.dot_general(gt_blk, a_ref[0],
                                      (((1,), (0,)), ((), ())), **_DOT)

    @pl.when(r == nr - 1)
    def _fini():
        add1 = jnp.where(diag_ref[0, 0] == 0.0, 1.0, 0.0)
        dis2 = _dis_of(deg2_ref[0, 0], diag_ref[0, 0])
        gt_all = (m1_ref[0, 0] * dis2)[None, :] * zw2t_ref[0]
        acc = h2t_ref[0] + add1[None, :] * gt_all
        h2t = jnp.maximum(dis2[None, :] * acc + b2_ref[0][:, None], 0.0)
        h2t_ref[0] = h2t
        p = p2_ref[0]
        pn = jax.lax.rsqrt(jnp.sum(p * p))
        s2_ref[0, 0] = jnp.tanh(jnp.sum(h2t * p[:, None], axis=0) * pn)


def _rank2_kernel(s2_ref, rank1_ref, m1_ref, h2t_ref, rank2_ref, vt_ref,
                  *, k2):
    t = pl.program_id(1)
    s_all = s2_ref[0, 0]
    r1_all = rank1_ref[0, 0]
    m_all = m1_ref[0, 0]
    tt = h2t_ref.shape[2]
    sl = pl.ds(t * tt, tt)
    s_i = s2_ref[0, 0, sl]
    r1_i = rank1_ref[0, 0, sl]
    m_i = m1_ref[0, 0, sl]
    gt = (s_all[None, :] > s_i[:, None]).astype(jnp.float32)
    tie = jnp.where((s_all[None, :] == s_i[:, None])
                    & (r1_all[None, :] < r1_i[:, None]), 1.0, 0.0)
    rank2 = jnp.sum(m_all[None, :] * (gt + tie), axis=1)
    valid = m_i * (rank2 < k2).astype(jnp.float32)
    rank2_ref[0, 0, sl] = jnp.where(valid > 0.0, rank2, -1.0)
    vt_ref[0] = h2t_ref[0] * (s_i * valid)[None, :]


def _scatter_kernel(rank2_ref, vt_ref, out_ref):
    q = pl.program_id(1)
    tq = out_ref.shape[1]
    r2 = rank2_ref[0, 0]
    n = r2.shape[0]
    qidx = (jax.lax.broadcasted_iota(jnp.int32, (n, tq), 1)
            + q * tq).astype(jnp.float32)
    p = jnp.where(r2[:, None] == qidx, 1.0, 0.0)
    outt = jax.lax.dot_general(vt_ref[0], p,
                               (((1,), (0,)), ((), ())), **_DOT)
    out_ref[0] = outt.T


def kernel(x, A, W1, b1, p1, W2, b2, p2):
    B, N, D0 = x.shape
    D1 = W1.shape[1]
    D2 = W2.shape[1]
    k1 = int(math.ceil(0.8 * N))
    k2 = int(math.ceil(0.5 * k1))
    nr = N // RB
    nt = N // TT
    f32 = jnp.float32
    b1r, p1r = b1.reshape(1, D1), p1.reshape(1, D1)
    b2r, p2r = b2.reshape(1, D2), p2.reshape(1, D2)

    full_n = pl.BlockSpec((1, 1, N), lambda b, r: (b, 0, 0))
    full_d1n = pl.BlockSpec((1, D1, N), lambda b, r: (b, 0, 0))
    full_d2n = pl.BlockSpec((1, D2, N), lambda b, r: (b, 0, 0))
    a_blk = pl.BlockSpec((1, RB, N), lambda b, r: (b, r, 0))
    small1 = pl.BlockSpec((1, D1), lambda b, r: (0, 0))
    small2 = pl.BlockSpec((1, D2), lambda b, r: (0, 0))

    deg1raw, diag, xwt = pl.pallas_call(
        _pass1_kernel,
        grid=(B, nr),
        in_specs=[pl.BlockSpec((1, RB, D0), lambda b, r: (b, r, 0)),
                  pl.BlockSpec((D0, D1), lambda b, r: (0, 0)),
                  a_blk],
        out_specs=[full_n, full_n,
                   pl.BlockSpec((1, D1, RB), lambda b, r: (b, 0, r))],
        out_shape=[jax.ShapeDtypeStruct((B, 1, N), f32),
                   jax.ShapeDtypeStruct((B, 1, N), f32),
                   jax.ShapeDtypeStruct((B, D1, N), f32)],
    )(x, W1, A)

    h1t, s1 = pl.pallas_call(
        functools.partial(_pass2_kernel, nr=nr),
        grid=(B, nr),
        in_specs=[a_blk, full_d1n, full_n, full_n, small1, small1],
        out_specs=[full_d1n, full_n],
        out_shape=[jax.ShapeDtypeStruct((B, D1, N), f32),
                   jax.ShapeDtypeStruct((B, 1, N), f32)],
    )(A, xwt, deg1raw, diag, b1r, p1r)

    rank1, m1, zw2t, deg2raw = pl.pallas_call(
        functools.partial(_pass3_kernel, k1=k1),
        grid=(B, nr),
        in_specs=[full_n, full_d1n,
                  pl.BlockSpec((D1, D2), lambda b, r: (0, 0)),
                  a_blk],
        out_specs=[full_n, full_n,
                   pl.BlockSpec((1, D2, RB), lambda b, r: (b, 0, r)),
                   full_n],
        out_shape=[jax.ShapeDtypeStruct((B, 1, N), f32),
                   jax.ShapeDtypeStruct((B, 1, N), f32),
                   jax.ShapeDtypeStruct((B, D2, N), f32),
                   jax.ShapeDtypeStruct((B, 1, N), f32)],
    )(s1, h1t, W2, A)

    h2t, s2 = pl.pallas_call(
        functools.partial(_pass4_kernel, nr=nr),
        grid=(B, nr),
        in_specs=[a_blk, full_d2n, full_n, full_n, full_n, small2, small2],
        out_specs=[full_d2n, full_n],
        out_shape=[jax.ShapeDtypeStruct((B, D2, N), f32),
                   jax.ShapeDtypeStruct((B, 1, N), f32)],
    )(A, zw2t, deg2raw, diag, m1, b2r, p2r)

    rank2, vt = pl.pallas_call(
        functools.partial(_rank2_kernel, k2=k2),
        grid=(B, nt),
        in_specs=[full_n, full_n, full_n,
                  pl.BlockSpec((1, D2, TT), lambda b, t: (b, 0, t))],
        out_specs=[full_n,
                   pl.BlockSpec((1, D2, TT), lambda b, t: (b, 0, t))],
        out_shape=[jax.ShapeDtypeStruct((B, 1, N), f32),
                   jax.ShapeDtypeStruct((B, D2, N), f32)],
    )(s2, rank1, m1, h2t)

    out = pl.pallas_call(
        _scatter_kernel,
        grid=(B, nt),
        in_specs=[full_n, full_d2n],
        out_specs=pl.BlockSpec((1, TT, D2), lambda b, q: (b, q, 0)),
        out_shape=jax.ShapeDtypeStruct((B, N, D2), f32),
    )(rank2, vt)
    return out


# speedup vs baseline: 2.2301x; 1.0972x over previous
import functools
import math

import jax
import jax.numpy as jnp
from jax.experimental import pallas as pl

RB = 512
TT = 512

_DOT = dict(preferred_element_type=jnp.float32,
            precision=jax.lax.Precision.HIGHEST)


def _dis_of(deg_raw, diag):
    deg = deg_raw + jnp.where(diag == 0.0, 1.0, 0.0)
    return jnp.where(deg > 0.0,
                     jax.lax.rsqrt(jnp.where(deg > 0.0, deg, 1.0)), 0.0)


def _pass1_kernel(x_ref, w1_ref, a_ref, deg_ref, diag_ref, xwt_ref):
    r = pl.program_id(1)
    a = a_ref[0]
    rb = a.shape[0]

    @pl.when(r == 0)
    def _init():
        deg_ref[0, 0] = jnp.zeros_like(deg_ref[0, 0])

    deg_ref[0, 0] += jnp.sum(a, axis=0)

    asq = a_ref[0, :, pl.ds(r * rb, rb)]
    ii = jax.lax.broadcasted_iota(jnp.int32, (rb, rb), 0)
    jj = jax.lax.broadcasted_iota(jnp.int32, (rb, rb), 1)
    diag_ref[0, 0, pl.ds(r * rb, rb)] = jnp.sum(
        jnp.where(ii == jj, asq, 0.0), axis=1)

    xw = jax.lax.dot_general(x_ref[0], w1_ref[...],
                             (((1,), (0,)), ((), ())), **_DOT)
    xwt_ref[0] = xw.T


def _pass2_kernel(a_ref, xwt_ref, deg_ref, diag_ref, b1_ref, p1_ref,
                  h1t_ref, s1_ref, *, nr):
    r = pl.program_id(1)
    rb = a_ref.shape[1]
    sl = pl.ds(r * rb, rb)
    dis1_blk = _dis_of(deg_ref[0, 0, sl], diag_ref[0, 0, sl])
    ut_blk = dis1_blk[None, :] * xwt_ref[0, :, sl]

    @pl.when(r == 0)
    def _init():
        h1t_ref[0] = jnp.zeros_like(h1t_ref[0])

    h1t_ref[0] += jax.lax.dot_general(ut_blk, a_ref[0],
                                      (((1,), (0,)), ((), ())), **_DOT)

    @pl.when(r == nr - 1)
    def _fini():
        add1 = jnp.where(diag_ref[0, 0] == 0.0, 1.0, 0.0)
        dis1 = _dis_of(deg_ref[0, 0], diag_ref[0, 0])
        ut_all = dis1[None, :] * xwt_ref[0]
        acc = h1t_ref[0] + add1[None, :] * ut_all
        h1t = jnp.maximum(dis1[None, :] * acc + b1_ref[0][:, None], 0.0)
        h1t_ref[0] = h1t
        p = p1_ref[0]
        pn = jax.lax.rsqrt(jnp.sum(p * p))
        s1_ref[0, 0] = jnp.tanh(jnp.sum(h1t * p[:, None], axis=0) * pn)


def _pass3_kernel(s1_ref, h1t_ref, w2_ref, a_ref,
                  rank_ref, m1_ref, zw2t_ref, deg2_ref, *, k1):
    r = pl.program_id(1)
    rb = a_ref.shape[1]
    sl = pl.ds(r * rb, rb)
    s_all = s1_ref[0, 0]
    s_i = s1_ref[0, 0, sl]
    n = s_all.shape[0]
    jidx = jax.lax.broadcasted_iota(jnp.int32, (rb, n), 1)
    iidx = jax.lax.broadcasted_iota(jnp.int32, (rb, n), 0) + r * rb
    gt = (s_all[None, :] > s_i[:, None]).astype(jnp.float32)
    tie = jnp.where((s_all[None, :] == s_i[:, None]) & (jidx < iidx), 1.0, 0.0)
    rank = jnp.sum(gt + tie, axis=1)
    m1 = (rank < k1).astype(jnp.float32)
    rank_ref[0, 0, sl] = rank
    m1_ref[0, 0, sl] = m1
    zt = h1t_ref[0, :, sl] * (s_i * m1)[None, :]
    zw2t_ref[0] = jax.lax.dot_general(w2_ref[...], zt,
                                      (((0,), (0,)), ((), ())), **_DOT)

    @pl.when(r == 0)
    def _init():
        deg2_ref[0, 0] = jnp.zeros_like(deg2_ref[0, 0])

    deg2_ref[0, 0] += jnp.sum(jnp.where(m1[:, None] > 0.0, a_ref[0], 0.0),
                              axis=0)


def _pass4_kernel(a_ref, zw2t_ref, deg2_ref, diag_ref, m1_ref, b2_ref, p2_ref,
                  h2t_ref, s2_ref, *, nr):
    r = pl.program_id(1)
    rb = a_ref.shape[1]
    sl = pl.ds(r * rb, rb)
    gs_blk = m1_ref[0, 0, sl] * _dis_of(deg2_ref[0, 0, sl], diag_ref[0, 0, sl])
    gt_blk = gs_blk[None, :] * zw2t_ref[0, :, sl]

    @pl.when(r == 0)
    def _init():
        h2t_ref[0] = jnp.zeros_like(h2t_ref[0])

    h2t_ref[0] += jax.lax.dot_general(gt_blk, a_ref[0],
                                      (((1,), (0,)), ((), ())), **_DOT)

    @pl.when(r == nr - 1)
    def _fini():
        add1 = jnp.where(diag_ref[0, 0] == 0.0, 1.0, 0.0)
        dis2 = _dis_of(deg2_ref[0, 0], diag_ref[0, 0])
        gt_all = (m1_ref[0, 0] * dis2)[None, :] * zw2t_ref[0]
        acc = h2t_ref[0] + add1[None, :] * gt_all
        h2t = jnp.maximum(dis2[None, :] * acc + b2_ref[0][:, None], 0.0)
        h2t_ref[0] = h2t
        p = p2_ref[0]
        pn = jax.lax.rsqrt(jnp.sum(p * p))
        s2_ref[0, 0] = jnp.tanh(jnp.sum(h2t * p[:, None], axis=0) * pn)


def _rank2_kernel(s2_ref, rank1_ref, m1_ref, h2t_ref, rank2_ref, vt_ref,
                  *, k2):
    t = pl.program_id(1)
    s_all = s2_ref[0, 0]
    r1_all = rank1_ref[0, 0]
    m_all = m1_ref[0, 0]
    tt = h2t_ref.shape[2]
    sl = pl.ds(t * tt, tt)
    s_i = s2_ref[0, 0, sl]
    r1_i = rank1_ref[0, 0, sl]
    m_i = m1_ref[0, 0, sl]
    gt = (s_all[None, :] > s_i[:, None]).astype(jnp.float32)
    tie = jnp.where((s_all[None, :] == s_i[:, None])
                    & (r1_all[None, :] < r1_i[:, None]), 1.0, 0.0)
    rank2 = jnp.sum(m_all[None, :] * (gt + tie), axis=1)
    valid = m_i * (rank2 < k2).astype(jnp.float32)
    rank2_ref[0, 0, sl] = jnp.where(valid > 0.0, rank2, -1.0)
    vt_ref[0] = h2t_ref[0] * (s_i * valid)[None, :]


def _scatter_kernel(rank2_ref, vt_ref, out_ref, *, k2):
    q = pl.program_id(1)
    tq = out_ref.shape[1]

    @pl.when(q * tq >= k2)
    def _zero():
        out_ref[0] = jnp.zeros_like(out_ref[0])

    @pl.when(q * tq < k2)
    def _dot():
        r2 = rank2_ref[0, 0]
        n = r2.shape[0]
        qidx = (jax.lax.broadcasted_iota(jnp.int32, (n, tq), 1)
                + q * tq).astype(jnp.float32)
        p = jnp.where(r2[:, None] == qidx, 1.0, 0.0)
        outt = jax.lax.dot_general(vt_ref[0], p,
                                   (((1,), (0,)), ((), ())), **_DOT)
        out_ref[0] = outt.T


def kernel(x, A, W1, b1, p1, W2, b2, p2):
    B, N, D0 = x.shape
    D1 = W1.shape[1]
    D2 = W2.shape[1]
    k1 = int(math.ceil(0.8 * N))
    k2 = int(math.ceil(0.5 * k1))
    nr = N // RB
    nt = N // TT
    f32 = jnp.float32
    b1r, p1r = b1.reshape(1, D1), p1.reshape(1, D1)
    b2r, p2r = b2.reshape(1, D2), p2.reshape(1, D2)

    full_n = pl.BlockSpec((1, 1, N), lambda b, r: (b, 0, 0))
    full_d1n = pl.BlockSpec((1, D1, N), lambda b, r: (b, 0, 0))
    full_d2n = pl.BlockSpec((1, D2, N), lambda b, r: (b, 0, 0))
    a_blk = pl.BlockSpec((1, RB, N), lambda b, r: (b, r, 0))
    small1 = pl.BlockSpec((1, D1), lambda b, r: (0, 0))
    small2 = pl.BlockSpec((1, D2), lambda b, r: (0, 0))

    deg1raw, diag, xwt = pl.pallas_call(
        _pass1_kernel,
        grid=(B, nr),
        in_specs=[pl.BlockSpec((1, RB, D0), lambda b, r: (b, r, 0)),
                  pl.BlockSpec((D0, D1), lambda b, r: (0, 0)),
                  a_blk],
        out_specs=[full_n, full_n,
                   pl.BlockSpec((1, D1, RB), lambda b, r: (b, 0, r))],
        out_shape=[jax.ShapeDtypeStruct((B, 1, N), f32),
                   jax.ShapeDtypeStruct((B, 1, N), f32),
                   jax.ShapeDtypeStruct((B, D1, N), f32)],
    )(x, W1, A)

    h1t, s1 = pl.pallas_call(
        functools.partial(_pass2_kernel, nr=nr),
        grid=(B, nr),
        in_specs=[a_blk, full_d1n, full_n, full_n, small1, small1],
        out_specs=[full_d1n, full_n],
        out_shape=[jax.ShapeDtypeStruct((B, D1, N), f32),
                   jax.ShapeDtypeStruct((B, 1, N), f32)],
    )(A, xwt, deg1raw, diag, b1r, p1r)

    rank1, m1, zw2t, deg2raw = pl.pallas_call(
        functools.partial(_pass3_kernel, k1=k1),
        grid=(B, nr),
        in_specs=[full_n, full_d1n,
                  pl.BlockSpec((D1, D2), lambda b, r: (0, 0)),
                  a_blk],
        out_specs=[full_n, full_n,
                   pl.BlockSpec((1, D2, RB), lambda b, r: (b, 0, r)),
                   full_n],
        out_shape=[jax.ShapeDtypeStruct((B, 1, N), f32),
                   jax.ShapeDtypeStruct((B, 1, N), f32),
                   jax.ShapeDtypeStruct((B, D2, N), f32),
                   jax.ShapeDtypeStruct((B, 1, N), f32)],
    )(s1, h1t, W2, A)

    h2t, s2 = pl.pallas_call(
        functools.partial(_pass4_kernel, nr=nr),
        grid=(B, nr),
        in_specs=[a_blk, full_d2n, full_n, full_n, full_n, small2, small2],
        out_specs=[full_d2n, full_n],
        out_shape=[jax.ShapeDtypeStruct((B, D2, N), f32),
                   jax.ShapeDtypeStruct((B, 1, N), f32)],
    )(A, zw2t, deg2raw, diag, m1, b2r, p2r)

    rank2, vt = pl.pallas_call(
        functools.partial(_rank2_kernel, k2=k2),
        grid=(B, nt),
        in_specs=[full_n, full_n, full_n,
                  pl.BlockSpec((1, D2, TT), lambda b, t: (b, 0, t))],
        out_specs=[full_n,
                   pl.BlockSpec((1, D2, TT), lambda b, t: (b, 0, t))],
        out_shape=[jax.ShapeDtypeStruct((B, 1, N), f32),
                   jax.ShapeDtypeStruct((B, D2, N), f32)],
    )(s2, rank1, m1, h2t)

    out = pl.pallas_call(
        functools.partial(_scatter_kernel, k2=k2),
        grid=(B, nt),
        in_specs=[full_n, full_d2n],
        out_specs=pl.BlockSpec((1, TT, D2), lambda b, q: (b, q, 0)),
        out_shape=jax.ShapeDtypeStruct((B, N, D2), f32),
    )(rank2, vt)
    return out
